# baseline (speedup 1.0000x reference)
import functools

import jax
import jax.numpy as jnp
import numpy as np
from jax import lax
from jax.experimental import pallas as pl
from jax.experimental.pallas import tpu as pltpu

_HEIGHTS = [11, 16, 23, 33, 48, 68, 97, 139, 198, 283]
_W = 16
_STRIDE = 16
_A = len(_HEIGHTS)


def _band_structure(H, W):
    h_img = H * _STRIDE
    counts = []
    for i in range(H):
        cy = (i + 0.5) * _STRIDE
        c = 0
        for h in _HEIGHTS:
            if cy - 0.5 * h >= 0 and cy + 0.5 * h <= h_img:
                c += 1
            else:
                break
        counts.append(c)
    for j in range(W):
        cx = (j + 0.5) * _STRIDE
        assert cx - 0.5 * _W >= 0 and cx + 0.5 * _W <= W * _STRIDE
    bands = []
    s = 0
    i = 0
    while i < H:
        r0 = i
        c = counts[i]
        while i < H and counts[i] == c:
            i += 1
        nr = i - r0
        bands.append((r0, nr, c, s))
        s += W * c * nr
    return bands, s


def _gen_body(bands, B, H, W, num_valid, anchors_ref, idx_ref):
    for (r0, nr, c, s0) in bands:
        size = W * c * nr
        slot = lax.broadcasted_iota(jnp.int32, (size,), 0)
        i_s = r0 + slot // (W * c)
        j_s = (slot // c) % W
        a_s = slot % c
        iv = (i_s * W + j_s) * _A + a_s
        t = lax.broadcasted_iota(jnp.int32, (4 * size,), 0)
        i_t = r0 + t // (4 * W * c)
        j_t = (t // (4 * c)) % W
        a_t = (t // 4) % c
        coord = t % 4
        cy = (i_t * _STRIDE + _STRIDE // 2).astype(jnp.float32)
        cx = (j_t * _STRIDE + _STRIDE // 2).astype(jnp.float32)
        half_h = jnp.full((4 * size,), 0.5 * _HEIGHTS[0], jnp.float32)
        for k in range(1, c):
            half_h = jnp.where(a_t == k, 0.5 * _HEIGHTS[k], half_h)
        hi_half = coord >= 2
        vy = cy + jnp.where(hi_half, half_h, -half_h)
        vx = cx + jnp.where(hi_half, 0.5 * _W, -0.5 * _W)
        v = jnp.where((coord & 1) == 1, vx, vy)
        for b in range(B):
            anchors_ref[b, pl.ds(4 * s0, 4 * size)] = v
            idx_ref[b, pl.ds(s0, size)] = iv


def kernel(features):
    B, H, W, C = features.shape
    bands, num_valid = _band_structure(H, W)
    anchors_flat, idx = pl.pallas_call(
        functools.partial(_gen_body, bands, B, H, W, num_valid),
        out_shape=[
            jax.ShapeDtypeStruct((B, 4 * num_valid), jnp.float32),
            jax.ShapeDtypeStruct((B, num_valid), jnp.int32),
        ],
    )()
    return (anchors_flat.reshape(B, num_valid, 4), idx)


# speedup vs baseline: 1.1646x; 1.1646x over previous
import functools

import jax
import jax.numpy as jnp
from jax import lax
from jax.experimental import pallas as pl
from jax.experimental.pallas import tpu as pltpu
from jax.experimental.pallas import tpu_sc as plsc

_HEIGHTS = [11, 16, 23, 33, 48, 68, 97, 139, 198, 283]
_W = 16
_STRIDE = 16
_A = len(_HEIGHTS)

_NC, _NS, _L = 2, 16, 16
_NW = _NC * _NS


def _row_bounds(H):
    img = H * _STRIDE
    bounds = []
    for h in _HEIGHTS:
        lo = 0
        while (lo + 0.5) * _STRIDE - 0.5 * h < 0:
            lo += 1
        hi = H - 1
        while (hi + 0.5) * _STRIDE + 0.5 * h > img:
            hi -= 1
        bounds.append((lo, hi))
    return bounds


def _bands(H, W):
    bounds = _row_bounds(H)
    counts = [sum(1 for lo, hi in bounds if lo <= i <= hi) for i in range(H)]
    bands, s, i = [], 0, 0
    while i < H:
        r0, c = i, counts[i]
        while i < H and counts[i] == c:
            i += 1
        bands.append((r0, i - r0, c, s))
        s += W * c * (i - r0)
    return bands, s


def _sc_body(H, W, num_valid, chunks, anch_out, idx_out, htab_ref, anch_buf,
             idx_buf):
    wid = lax.axis_index("s") * _NC + lax.axis_index("c")
    bounds = _row_bounds(H)
    bands, _ = _bands(H, W)
    distinct_counts = sorted({b[2] for b in bands})
    lanes = lax.broadcasted_iota(jnp.int32, (_L,), 0)

    def cum_slots(r):
        t = jnp.int32(0)
        for (lo, hi) in bounds:
            t = t + jnp.maximum(jnp.minimum(r, hi + 1) - lo, 0)
        return t * W

    def row_scalars(r):
        c0 = cum_slots(r)
        c1 = cum_slots(r + 1)
        cnt = (c1 - c0) >> 6
        rcp = jnp.float32(0.0)
        for c in distinct_counts:
            rcp = jnp.where(cnt == c, jnp.float32(1.0) / jnp.float32(c), rcp)
        return c0, cnt, rcp

    hv = jnp.full((_L,), 0.0, jnp.float32)
    for k, h in enumerate(_HEIGHTS):
        hv = jnp.where(lanes == k, jnp.float32(h), hv)
    htab_ref[pl.ds(0, _L)] = hv

    chunk_n, chunk_v, stride_a, stride_b, n_a = chunks
    n0 = jnp.where(wid <= n_a, stride_a * wid,
                   stride_a * n_a + stride_b * (wid - n_a)).astype(jnp.int32)
    base4 = lanes * 4

    counts = [b[2] for b in bands for _ in range(b[1])]
    row_off = [0]
    for c in counts:
        row_off.append(row_off[-1] + W * c)
    i0 = jnp.int32(0)
    for thresh in row_off[1:-1]:
        i0 = jnp.where(n0 >= thresh, i0 + 1, i0)
    cA, cntA, rcpA = row_scalars(i0)
    cB, cntB, rcpB = row_scalars(i0 + 1)

    def step(v, carry):
        i, c0, cnt0, rcp0, c1, cnt1, rcp1 = carry
        n = n0 + _L * v + lanes
        cross = n >= c1
        iv = jnp.where(cross, i + 1, i)
        m = n - jnp.where(cross, c1, c0)
        cv = jnp.where(cross, cnt1, cnt0)
        rc = jnp.where(cross, rcp1, rcp0)
        j = ((m.astype(jnp.float32) + 0.5) * rc).astype(jnp.int32)
        a = m - j * cv
        idx_buf[pl.ds(_L * v, _L)] = iv * (W * _A) + j * _A + a
        cy = (iv * _STRIDE + _STRIDE // 2).astype(jnp.float32)
        cx = (j * _STRIDE + _STRIDE // 2).astype(jnp.float32)
        hh = jnp.full((_L,), 0.0, jnp.float32)
        for kk, h in enumerate(_HEIGHTS):
            hh = jnp.where(a == kk, jnp.float32(0.5 * h), hh)
        pos = base4 + (4 * _L) * v
        plsc.store_scatter(anch_buf, [pos], cy - hh)
        plsc.store_scatter(anch_buf, [pos + 1], cx - 0.5 * _W)
        plsc.store_scatter(anch_buf, [pos + 2], cy + hh)
        plsc.store_scatter(anch_buf, [pos + 3], cx + 0.5 * _W)

        def advance(_):
            nc0, ncnt0, nrcp0 = c1, cnt1, rcp1
            nc1, ncnt1, nrcp1 = row_scalars(i + 2)
            return (i + 1, nc0, ncnt0, nrcp0, nc1, ncnt1, nrcp1)

        n_end = n0 + _L * v + (_L - 1)
        return lax.cond(n_end >= c1, advance, lambda _: carry, 0)

    lax.fori_loop(0, chunk_v, step, (i0, cA, cntA, rcpA, cB, cntB, rcpB))

    B = anch_out.shape[0] // (4 * num_valid)
    for b in range(B):
        pltpu.sync_copy(idx_buf.at[pl.ds(0, chunk_n)],
                        idx_out.at[pl.ds(b * num_valid + n0, chunk_n)])
        pltpu.sync_copy(anch_buf.at[pl.ds(0, 4 * chunk_n)],
                        anch_out.at[pl.ds(4 * (b * num_valid + n0),
                                          4 * chunk_n)])


def kernel(features):
    B, H, W, C = features.shape
    _, num_valid = _bands(H, W)

    chunk_v = -(-num_valid // (_NW * _L))
    chunk_n = chunk_v * _L
    span = num_valid - chunk_n
    stride_b = (span // (_NW - 1)) // 8 * 8
    n_a = (span - stride_b * (_NW - 1)) // 8
    stride_a = stride_b + 8
    assert 0 <= n_a <= _NW - 1 and stride_a * n_a + stride_b * (_NW - 1 - n_a) == span
    assert chunk_n % 8 == 0
    chunks = (chunk_n, chunk_v, stride_a, stride_b, n_a)
    big_n = chunk_n

    mesh = plsc.VectorSubcoreMesh(core_axis_name="c", subcore_axis_name="s")
    k = functools.partial(
        pl.kernel,
        mesh=mesh,
        compiler_params=pltpu.CompilerParams(needs_layout_passes=False,
                                             use_tc_tiling_on_sc=False),
        out_type=[
            jax.ShapeDtypeStruct((B * 4 * num_valid,), jnp.float32),
            jax.ShapeDtypeStruct((B * num_valid,), jnp.int32),
        ],
        scratch_types=[
            pltpu.VMEM((_L,), jnp.float32),
            pltpu.VMEM((4 * big_n,), jnp.float32),
            pltpu.VMEM((big_n,), jnp.int32),
        ],
    )(functools.partial(_sc_body, H, W, num_valid, chunks))
    anchors_flat, idx_flat = k()
    return (anchors_flat.reshape(B, num_valid, 4),
            idx_flat.reshape(B, num_valid))


# speedup vs baseline: 1.4650x; 1.2579x over previous
import functools

import jax
import jax.numpy as jnp
from jax import lax
from jax.experimental import pallas as pl
from jax.experimental.pallas import tpu as pltpu
from jax.experimental.pallas import tpu_sc as plsc

_HEIGHTS = [11, 16, 23, 33, 48, 68, 97, 139, 198, 283]
_W = 16
_STRIDE = 16
_A = len(_HEIGHTS)

_NC, _NS, _L = 2, 16, 16
_NW = _NC * _NS


def _row_bounds(H):
    img = H * _STRIDE
    bounds = []
    for h in _HEIGHTS:
        lo = 0
        while (lo + 0.5) * _STRIDE - 0.5 * h < 0:
            lo += 1
        hi = H - 1
        while (hi + 0.5) * _STRIDE + 0.5 * h > img:
            hi -= 1
        bounds.append((lo, hi))
    return bounds


def _bands(H, W):
    bounds = _row_bounds(H)
    counts = [sum(1 for lo, hi in bounds if lo <= i <= hi) for i in range(H)]
    bands, s, i = [], 0, 0
    while i < H:
        r0, c = i, counts[i]
        while i < H and counts[i] == c:
            i += 1
        bands.append((r0, i - r0, c, s))
        s += W * c * (i - r0)
    return bands, s


def _sc_body(H, W, num_valid, chunks, anch_out, idx_out, htab_ref, anch_buf,
             idx_buf):
    wid = lax.axis_index("s") * _NC + lax.axis_index("c")
    bounds = _row_bounds(H)
    bands, _ = _bands(H, W)
    distinct_counts = sorted({b[2] for b in bands})
    lanes = lax.broadcasted_iota(jnp.int32, (_L,), 0)

    def cum_slots(r):
        t = jnp.int32(0)
        for (lo, hi) in bounds:
            t = t + jnp.maximum(jnp.minimum(r, hi + 1) - lo, 0)
        return t * W

    def row_scalars(r):
        c0 = cum_slots(r)
        c1 = cum_slots(r + 1)
        cnt = (c1 - c0) >> 6
        rcp = jnp.float32(0.0)
        for c in distinct_counts:
            rcp = jnp.where(cnt == c, jnp.float32(1.0) / jnp.float32(c), rcp)
        return c0, cnt, rcp

    hv = jnp.full((_L,), 0.0, jnp.float32)
    for k, h in enumerate(_HEIGHTS):
        hv = jnp.where(lanes == k, jnp.float32(h), hv)
    htab_ref[pl.ds(0, _L)] = hv

    chunk_n, chunk_v, stride_a, stride_b, n_a = chunks
    n0 = jnp.where(wid <= n_a, stride_a * wid,
                   stride_a * n_a + stride_b * (wid - n_a)).astype(jnp.int32)
    base4 = lanes * 4

    counts = [b[2] for b in bands for _ in range(b[1])]
    row_off = [0]
    for c in counts:
        row_off.append(row_off[-1] + W * c)
    i0 = jnp.int32(0)
    for thresh in row_off[1:-1]:
        i0 = jnp.where(n0 >= thresh, i0 + 1, i0)
    cA, cntA, rcpA = row_scalars(i0)
    cB, cntB, rcpB = row_scalars(i0 + 1)

    def step(v, carry):
        i, c0, cnt0, rcp0, c1, cnt1, rcp1 = carry
        n = n0 + _L * v + lanes
        cross = n >= c1
        iv = jnp.where(cross, i + 1, i)
        m = n - jnp.where(cross, c1, c0)
        cv = jnp.where(cross, cnt1, cnt0)
        rc = jnp.where(cross, rcp1, rcp0)
        j = ((m.astype(jnp.float32) + 0.5) * rc).astype(jnp.int32)
        a = m - j * cv
        idx_buf[pl.ds(_L * v, _L)] = iv * (W * _A) + j * _A + a
        cy = (iv * _STRIDE + _STRIDE // 2).astype(jnp.float32)
        cx = (j * _STRIDE + _STRIDE // 2).astype(jnp.float32)
        hh = jnp.full((_L,), 0.0, jnp.float32)
        for kk, h in enumerate(_HEIGHTS):
            hh = jnp.where(a == kk, jnp.float32(0.5 * h), hh)
        row = lanes + _L * v
        for cc, val in enumerate((cy - hh, cx - 0.5 * _W,
                                  cy + hh, cx + 0.5 * _W)):
            plsc.store_scatter(anch_buf, [row, jnp.full((_L,), cc, jnp.int32)],
                               val)

        def advance(_):
            nc0, ncnt0, nrcp0 = c1, cnt1, rcp1
            nc1, ncnt1, nrcp1 = row_scalars(i + 2)
            return (i + 1, nc0, ncnt0, nrcp0, nc1, ncnt1, nrcp1)

        n_end = n0 + _L * v + (_L - 1)
        return lax.cond(n_end >= c1, advance, lambda _: carry, 0)

    lax.fori_loop(0, chunk_v, step, (i0, cA, cntA, rcpA, cB, cntB, rcpB))

    B = anch_out.shape[0]
    for b in range(B):
        pltpu.sync_copy(idx_buf, idx_out.at[b, pl.ds(n0, chunk_n)])
        pltpu.sync_copy(anch_buf, anch_out.at[b, pl.ds(n0, chunk_n)])


def kernel(features):
    B, H, W, C = features.shape
    _, num_valid = _bands(H, W)

    chunk_v = -(-num_valid // (_NW * _L))
    chunk_n = chunk_v * _L
    span = num_valid - chunk_n
    stride_b = (span // (_NW - 1)) // 8 * 8
    n_a = (span - stride_b * (_NW - 1)) // 8
    stride_a = stride_b + 8
    assert 0 <= n_a <= _NW - 1 and stride_a * n_a + stride_b * (_NW - 1 - n_a) == span
    assert chunk_n % 8 == 0
    chunks = (chunk_n, chunk_v, stride_a, stride_b, n_a)
    big_n = chunk_n

    mesh = plsc.VectorSubcoreMesh(core_axis_name="c", subcore_axis_name="s")
    k = functools.partial(
        pl.kernel,
        mesh=mesh,
        compiler_params=pltpu.CompilerParams(needs_layout_passes=False,
                                             use_tc_tiling_on_sc=False),
        out_type=[
            jax.ShapeDtypeStruct((B, num_valid, 4), jnp.float32),
            jax.ShapeDtypeStruct((B, num_valid), jnp.int32),
        ],
        scratch_types=[
            pltpu.VMEM((_L,), jnp.float32),
            pltpu.VMEM((big_n, 4), jnp.float32),
            pltpu.VMEM((big_n,), jnp.int32),
        ],
    )(functools.partial(_sc_body, H, W, num_valid, chunks))
    anchors, idx = k()
    return (anchors, idx)


# speedup vs baseline: 3.4217x; 2.3357x over previous
import functools

import jax
import jax.numpy as jnp
from jax import lax
from jax.experimental import pallas as pl
from jax.experimental.pallas import tpu as pltpu
from jax.experimental.pallas import tpu_sc as plsc

_HEIGHTS = [11, 16, 23, 33, 48, 68, 97, 139, 198, 283]
_W = 16
_STRIDE = 16
_A = len(_HEIGHTS)

_NC, _NS, _L = 2, 16, 16
_NW = _NC * _NS


def _row_bounds(H):
    img = H * _STRIDE
    bounds = []
    for h in _HEIGHTS:
        lo = 0
        while (lo + 0.5) * _STRIDE - 0.5 * h < 0:
            lo += 1
        hi = H - 1
        while (hi + 0.5) * _STRIDE + 0.5 * h > img:
            hi -= 1
        bounds.append((lo, hi))
    return bounds


def _bands(H, W):
    bounds = _row_bounds(H)
    counts = [sum(1 for lo, hi in bounds if lo <= i <= hi) for i in range(H)]
    bands, s, i = [], 0, 0
    while i < H:
        r0, c = i, counts[i]
        while i < H and counts[i] == c:
            i += 1
        bands.append((r0, i - r0, c, s))
        s += W * c * (i - r0)
    return bands, s


def _sc_body(H, W, num_valid, chunks, anch_out, idx_out, htab_ref, anch_buf,
             idx_buf):
    wid = lax.axis_index("s") * _NC + lax.axis_index("c")
    bounds = _row_bounds(H)
    bands, _ = _bands(H, W)
    distinct_counts = sorted({b[2] for b in bands})
    lanes = lax.broadcasted_iota(jnp.int32, (_L,), 0)

    def cum_slots(r):
        t = jnp.int32(0)
        for (lo, hi) in bounds:
            t = t + jnp.maximum(jnp.minimum(r, hi + 1) - lo, 0)
        return t * W

    def row_scalars(r):
        c0 = cum_slots(r)
        c1 = cum_slots(r + 1)
        cnt = (c1 - c0) >> 6
        rcp = jnp.float32(0.0)
        for c in distinct_counts:
            rcp = jnp.where(cnt == c, jnp.float32(1.0) / jnp.float32(c), rcp)
        return c0, cnt, rcp

    hv = jnp.full((_L,), 0.0, jnp.float32)
    for k, h in enumerate(_HEIGHTS):
        hv = jnp.where(lanes == k, jnp.float32(h), hv)
    htab_ref[pl.ds(0, _L)] = hv

    chunk_n, chunk_v, stride_a, stride_b, n_a = chunks
    n0 = jnp.where(wid <= n_a, stride_a * wid,
                   stride_a * n_a + stride_b * (wid - n_a)).astype(jnp.int32)
    base4 = lanes * 4

    counts = [b[2] for b in bands for _ in range(b[1])]
    row_off = [0]
    for c in counts:
        row_off.append(row_off[-1] + W * c)
    i0 = jnp.int32(0)
    for thresh in row_off[1:-1]:
        i0 = jnp.where(n0 >= thresh, i0 + 1, i0)
    cA, cntA, rcpA = row_scalars(i0)
    cB, cntB, rcpB = row_scalars(i0 + 1)

    def step(v, carry):
        i, c0, cnt0, rcp0, c1, cnt1, rcp1 = carry
        n = n0 + _L * v + lanes
        cross = n >= c1
        iv = jnp.where(cross, i + 1, i)
        m = n - jnp.where(cross, c1, c0)
        cv = jnp.where(cross, cnt1, cnt0)
        rc = jnp.where(cross, rcp1, rcp0)
        j = ((m.astype(jnp.float32) + 0.5) * rc).astype(jnp.int32)
        a = m - j * cv
        idx_buf[pl.ds(_L * v, _L)] = iv * (W * _A) + j * _A + a
        cy = (iv * _STRIDE + _STRIDE // 2).astype(jnp.float32)
        cx = (j * _STRIDE + _STRIDE // 2).astype(jnp.float32)
        hh = jnp.full((_L,), 0.0, jnp.float32)
        for kk, h in enumerate(_HEIGHTS):
            hh = jnp.where(a == kk, jnp.float32(0.5 * h), hh)
        row = lanes + _L * v
        for cc, val in enumerate((cy - hh, cx - 0.5 * _W,
                                  cy + hh, cx + 0.5 * _W)):
            plsc.store_scatter(anch_buf, [row, jnp.full((_L,), cc, jnp.int32)],
                               val)

        def advance(_):
            nc0, ncnt0, nrcp0 = c1, cnt1, rcp1
            nc1, ncnt1, nrcp1 = row_scalars(i + 2)
            return (i + 1, nc0, ncnt0, nrcp0, nc1, ncnt1, nrcp1)

        n_end = n0 + _L * v + (_L - 1)
        return lax.cond(n_end >= c1, advance, lambda _: carry, 0)

    lax.fori_loop(0, chunk_v, step, (i0, cA, cntA, rcpA, cB, cntB, rcpB))

    pltpu.sync_copy(idx_buf, idx_out.at[pl.ds(n0, chunk_n)])
    pltpu.sync_copy(anch_buf, anch_out.at[pl.ds(n0, chunk_n)])


def kernel(features):
    B, H, W, C = features.shape
    _, num_valid = _bands(H, W)

    chunk_v = -(-num_valid // (_NW * _L))
    chunk_n = chunk_v * _L
    span = num_valid - chunk_n
    stride_b = (span // (_NW - 1)) // 8 * 8
    n_a = (span - stride_b * (_NW - 1)) // 8
    stride_a = stride_b + 8
    assert 0 <= n_a <= _NW - 1 and stride_a * n_a + stride_b * (_NW - 1 - n_a) == span
    assert chunk_n % 8 == 0
    chunks = (chunk_n, chunk_v, stride_a, stride_b, n_a)
    big_n = chunk_n

    mesh = plsc.VectorSubcoreMesh(core_axis_name="c", subcore_axis_name="s")
    k = functools.partial(
        pl.kernel,
        mesh=mesh,
        compiler_params=pltpu.CompilerParams(needs_layout_passes=False,
                                             use_tc_tiling_on_sc=False),
        out_type=[
            jax.ShapeDtypeStruct((num_valid, 4), jnp.float32),
            jax.ShapeDtypeStruct((num_valid,), jnp.int32),
        ],
        scratch_types=[
            pltpu.VMEM((_L,), jnp.float32),
            pltpu.VMEM((big_n, 4), jnp.float32),
            pltpu.VMEM((big_n,), jnp.int32),
        ],
    )(functools.partial(_sc_body, H, W, num_valid, chunks))
    anchors1, idx1 = k()
    anchors = jnp.broadcast_to(anchors1[None], (B, num_valid, 4))
    idx = jnp.broadcast_to(idx1[None], (B, num_valid))
    return (anchors, idx)
